# Initial kernel scaffold; baseline (speedup 1.0000x reference)
import functools

import jax
import jax.numpy as jnp
from jax import lax
from jax.experimental import pallas as pl
from jax.experimental.pallas import tpu as pltpu
from jax.experimental.pallas import tpu_sc as plsc

T = 2048
D = 768
F = 2048
E = 8
K = 2

B = 256
P = T * K
P_PAD = P + E * B
G = P_PAD // B

NC = 2
NS = 16
NW = NC * NS


def _make_sc_row_gather(n_rows: int, chunk: int):
    n_per_w = n_rows // NW
    n_chunks = n_per_w // chunk
    assert n_per_w % chunk == 0 and chunk % 8 == 0

    mesh = plsc.VectorSubcoreMesh(core_axis_name="c", subcore_axis_name="s")

    @functools.partial(
        pl.kernel,
        mesh=mesh,
        out_type=jax.ShapeDtypeStruct((n_rows, D), jnp.float32),
        scratch_types=[
            pltpu.VMEM((chunk,), jnp.int32),
            pltpu.VMEM((chunk, D), jnp.float32),
            pltpu.SemaphoreType.DMA,
        ],
    )
    def gather_kernel(table_hbm, idx_hbm, out_hbm, idx_v, rows_v, sem):
        wid = lax.axis_index("s") * NC + lax.axis_index("c")
        base = wid * n_per_w
        for c in range(n_chunks):
            off = base + c * chunk
            pltpu.sync_copy(idx_hbm.at[pl.ds(off, chunk)], idx_v)
            pltpu.async_copy(table_hbm.at[idx_v], rows_v, sem).wait()
            pltpu.sync_copy(rows_v, out_hbm.at[pl.ds(off, chunk)])

    return gather_kernel


_gather_dispatch = _make_sc_row_gather(P_PAD, 96)
_gather_combine = _make_sc_row_gather(P, 128)


def _ffn_body(blk_ref, eid_ref, x_ref, w0_ref, w1_ref, w2_ref, rw_ref, o_ref):
    s = pl.program_id(0)

    @pl.when(blk_ref[s] == s)
    def _():
        xb = x_ref[...]
        a = jnp.dot(xb, w0_ref[0], preferred_element_type=jnp.float32)
        b = jnp.dot(xb, w1_ref[0], preferred_element_type=jnp.float32)
        h = (a * jax.nn.sigmoid(a)) * b
        y = jnp.dot(h, w2_ref[0], preferred_element_type=jnp.float32)
        o_ref[...] = y * rw_ref[...]


_ffn_grid_spec = pltpu.PrefetchScalarGridSpec(
    num_scalar_prefetch=2,
    grid=(G,),
    in_specs=[
        pl.BlockSpec((B, D), lambda s, blk, eid: (blk[s], 0)),
        pl.BlockSpec((1, D, F), lambda s, blk, eid: (eid[s], 0, 0)),
        pl.BlockSpec((1, D, F), lambda s, blk, eid: (eid[s], 0, 0)),
        pl.BlockSpec((1, F, D), lambda s, blk, eid: (eid[s], 0, 0)),
        pl.BlockSpec((B, 1), lambda s, blk, eid: (blk[s], 0)),
    ],
    out_specs=pl.BlockSpec((B, D), lambda s, blk, eid: (blk[s], 0)),
)


def kernel(x, w0, w1, w2, selected_experts, routing_weights):
    e_flat = selected_experts.reshape(P).astype(jnp.int32)
    rw_flat = routing_weights.reshape(P)

    order = jnp.argsort(e_flat, stable=True).astype(jnp.int32)
    sorted_e = jnp.take(e_flat, order)
    counts = jnp.bincount(e_flat, length=E).astype(jnp.int32)
    csum = jnp.cumsum(counts)
    group_start = csum - counts
    padded_counts = ((counts + B - 1) // B) * B
    pcsum = jnp.cumsum(padded_counts)
    pad_start = pcsum - padded_counts
    pp = (jnp.arange(P, dtype=jnp.int32)
          - jnp.take(group_start, sorted_e) + jnp.take(pad_start, sorted_e))
    tok_padded = jnp.zeros((P_PAD,), jnp.int32).at[pp].set(order // K)
    pos = jnp.zeros((P,), jnp.int32).at[order].set(pp)
    rw_padded = jnp.zeros((P_PAD,), jnp.float32).at[pp].set(
        jnp.take(rw_flat, order))

    nb = pcsum[-1] // B
    s_ids = jnp.arange(G, dtype=jnp.int32)
    blk = jnp.minimum(s_ids, nb - 1).astype(jnp.int32)
    eid = jnp.searchsorted(pcsum, blk * B, side="right").astype(jnp.int32)

    x_padded = _gather_dispatch(x, tok_padded)

    y_scaled = pl.pallas_call(
        _ffn_body,
        grid_spec=_ffn_grid_spec,
        out_shape=jax.ShapeDtypeStruct((P_PAD, D), jnp.float32),
    )(blk, eid, x_padded, w0, w1, w2, rw_padded[:, None])

    out_flat = _gather_combine(y_scaled, pos)
    return out_flat.reshape(T, K, D)


# speedup vs baseline: 1.0876x; 1.0876x over previous
import functools

import jax
import jax.numpy as jnp
from jax import lax
from jax.experimental import pallas as pl
from jax.experimental.pallas import tpu as pltpu
from jax.experimental.pallas import tpu_sc as plsc

T = 2048
D = 768
F = 2048
E = 8
K = 2

B = 256
P = T * K
P_PAD = P + E * B
G = P_PAD // B

NC = 2
NS = 16
NW = NC * NS


@functools.lru_cache(maxsize=None)
def _make_sc_row_gather(n_rows: int, chunk: int):
    n_per_w = n_rows // NW
    n_chunks = n_per_w // chunk
    assert n_per_w % chunk == 0 and chunk % 8 == 0

    mesh = plsc.VectorSubcoreMesh(core_axis_name="c", subcore_axis_name="s",
                                  num_cores=NC, num_subcores=NS)

    @functools.partial(
        pl.kernel,
        mesh=mesh,
        out_type=jax.ShapeDtypeStruct((n_rows, D), jnp.float32),
        scratch_types=[
            pltpu.VMEM((chunk,), jnp.int32),
            pltpu.VMEM((chunk, D), jnp.float32),
            pltpu.SemaphoreType.DMA,
        ],
    )
    def gather_kernel(table_hbm, idx_hbm, out_hbm, idx_v, rows_v, sem):
        wid = lax.axis_index("s") * NC + lax.axis_index("c")
        base = wid * n_per_w
        for c in range(n_chunks):
            off = base + c * chunk
            pltpu.sync_copy(idx_hbm.at[pl.ds(off, chunk)], idx_v)
            pltpu.async_copy(table_hbm.at[idx_v], rows_v, sem).wait()
            pltpu.sync_copy(rows_v, out_hbm.at[pl.ds(off, chunk)])

    return gather_kernel


def _ffn_body(blk_ref, eid_ref, x_ref, w0_ref, w1_ref, w2_ref, rw_ref, o_ref):
    s = pl.program_id(0)

    @pl.when(blk_ref[s] == s)
    def _():
        xb = x_ref[...]
        a = jnp.dot(xb, w0_ref[0], preferred_element_type=jnp.float32)
        b = jnp.dot(xb, w1_ref[0], preferred_element_type=jnp.float32)
        h = (a * jax.nn.sigmoid(a)) * b
        y = jnp.dot(h, w2_ref[0], preferred_element_type=jnp.float32)
        o_ref[...] = y * rw_ref[...]


_ffn_grid_spec = pltpu.PrefetchScalarGridSpec(
    num_scalar_prefetch=2,
    grid=(G,),
    in_specs=[
        pl.BlockSpec((B, D), lambda s, blk, eid: (blk[s], 0)),
        pl.BlockSpec((1, D, F), lambda s, blk, eid: (eid[s], 0, 0)),
        pl.BlockSpec((1, D, F), lambda s, blk, eid: (eid[s], 0, 0)),
        pl.BlockSpec((1, F, D), lambda s, blk, eid: (eid[s], 0, 0)),
        pl.BlockSpec((B, 1), lambda s, blk, eid: (blk[s], 0)),
    ],
    out_specs=pl.BlockSpec((B, D), lambda s, blk, eid: (blk[s], 0)),
)


def kernel(x, w0, w1, w2, selected_experts, routing_weights):
    e_flat = selected_experts.reshape(P).astype(jnp.int32)
    rw_flat = routing_weights.reshape(P)

    order = jnp.argsort(e_flat, stable=True).astype(jnp.int32)
    sorted_e = jnp.take(e_flat, order)
    counts = jnp.bincount(e_flat, length=E).astype(jnp.int32)
    csum = jnp.cumsum(counts)
    group_start = csum - counts
    padded_counts = ((counts + B - 1) // B) * B
    pcsum = jnp.cumsum(padded_counts)
    pad_start = pcsum - padded_counts
    pp = (jnp.arange(P, dtype=jnp.int32)
          - jnp.take(group_start, sorted_e) + jnp.take(pad_start, sorted_e))
    tok_padded = jnp.zeros((P_PAD,), jnp.int32).at[pp].set(order // K)
    pos = jnp.zeros((P,), jnp.int32).at[order].set(pp)
    rw_padded = jnp.zeros((P_PAD,), jnp.float32).at[pp].set(
        jnp.take(rw_flat, order))

    nb = pcsum[-1] // B
    s_ids = jnp.arange(G, dtype=jnp.int32)
    blk = jnp.minimum(s_ids, nb - 1).astype(jnp.int32)
    eid = jnp.searchsorted(pcsum, blk * B, side="right").astype(jnp.int32)

    x_padded = _make_sc_row_gather(P_PAD, 96)(x, tok_padded)

    y_scaled = pl.pallas_call(
        _ffn_body,
        grid_spec=_ffn_grid_spec,
        out_shape=jax.ShapeDtypeStruct((P_PAD, D), jnp.float32),
    )(blk, eid, x_padded, w0, w1, w2, rw_padded[:, None])

    out_flat = _make_sc_row_gather(P, 128)(y_scaled, pos)
    return out_flat.reshape(T, K, D)


# speedup vs baseline: 1.1386x; 1.0469x over previous
import functools

import jax
import jax.numpy as jnp
from jax import lax
from jax.experimental import pallas as pl
from jax.experimental.pallas import tpu as pltpu
from jax.experimental.pallas import tpu_sc as plsc

T = 2048
D = 768
F = 2048
E = 8
K = 2

B = 256
P = T * K
P_PAD = P + E * B
G = P_PAD // B

NC = 2
NS = 16
NW = NC * NS


@functools.lru_cache(maxsize=None)
def _make_sc_row_gather(n_rows: int, chunk: int):
    n_per_w = n_rows // NW
    n_chunks = n_per_w // chunk
    assert n_per_w % chunk == 0 and chunk % 8 == 0

    mesh = plsc.VectorSubcoreMesh(core_axis_name="c", subcore_axis_name="s",
                                  num_cores=NC, num_subcores=NS)

    @functools.partial(
        pl.kernel,
        mesh=mesh,
        out_type=jax.ShapeDtypeStruct((n_rows, D), jnp.float32),
        scratch_types=[
            pltpu.VMEM((chunk,), jnp.int32),
            pltpu.VMEM((chunk, D), jnp.float32),
            pltpu.SemaphoreType.DMA,
        ],
    )
    def gather_kernel(table_hbm, idx_hbm, out_hbm, idx_v, rows_v, sem):
        wid = lax.axis_index("s") * NC + lax.axis_index("c")
        base = wid * n_per_w
        for c in range(n_chunks):
            off = base + c * chunk
            pltpu.sync_copy(idx_hbm.at[pl.ds(off, chunk)], idx_v)
            pltpu.async_copy(table_hbm.at[idx_v], rows_v, sem).wait()
            pltpu.sync_copy(rows_v, out_hbm.at[pl.ds(off, chunk)])

    return gather_kernel


def _ffn_body(blk_ref, eid_ref, x_ref, w0_ref, w1_ref, w2_ref, rw_ref, o_ref):
    s = pl.program_id(0)

    @pl.when(blk_ref[s] == s)
    def _():
        xb = x_ref[...].astype(jnp.bfloat16)
        a = jnp.dot(xb, w0_ref[0], preferred_element_type=jnp.float32)
        b = jnp.dot(xb, w1_ref[0], preferred_element_type=jnp.float32)
        h = ((a * jax.nn.sigmoid(a)) * b).astype(jnp.bfloat16)
        y = jnp.dot(h, w2_ref[0], preferred_element_type=jnp.float32)
        o_ref[...] = y * rw_ref[...]


_ffn_grid_spec = pltpu.PrefetchScalarGridSpec(
    num_scalar_prefetch=2,
    grid=(G,),
    in_specs=[
        pl.BlockSpec((B, D), lambda s, blk, eid: (blk[s], 0)),
        pl.BlockSpec((1, D, F), lambda s, blk, eid: (eid[s], 0, 0)),
        pl.BlockSpec((1, D, F), lambda s, blk, eid: (eid[s], 0, 0)),
        pl.BlockSpec((1, F, D), lambda s, blk, eid: (eid[s], 0, 0)),
        pl.BlockSpec((B, 1), lambda s, blk, eid: (blk[s], 0)),
    ],
    out_specs=pl.BlockSpec((B, D), lambda s, blk, eid: (blk[s], 0)),
)


def kernel(x, w0, w1, w2, selected_experts, routing_weights):
    e_flat = selected_experts.reshape(P).astype(jnp.int32)
    rw_flat = routing_weights.reshape(P)

    order = jnp.argsort(e_flat, stable=True).astype(jnp.int32)
    sorted_e = jnp.take(e_flat, order)
    counts = jnp.bincount(e_flat, length=E).astype(jnp.int32)
    csum = jnp.cumsum(counts)
    group_start = csum - counts
    padded_counts = ((counts + B - 1) // B) * B
    pcsum = jnp.cumsum(padded_counts)
    pad_start = pcsum - padded_counts
    pp = (jnp.arange(P, dtype=jnp.int32)
          - jnp.take(group_start, sorted_e) + jnp.take(pad_start, sorted_e))
    tok_padded = (jnp.arange(P_PAD, dtype=jnp.int32) % T).at[pp].set(order // K)
    pos = jnp.zeros((P,), jnp.int32).at[order].set(pp)
    rw_padded = jnp.zeros((P_PAD,), jnp.float32).at[pp].set(
        jnp.take(rw_flat, order))

    nb = pcsum[-1] // B
    s_ids = jnp.arange(G, dtype=jnp.int32)
    blk = jnp.minimum(s_ids, nb - 1).astype(jnp.int32)
    eid = jnp.searchsorted(pcsum, blk * B, side="right").astype(jnp.int32)

    x_padded = _make_sc_row_gather(P_PAD, 96)(x, tok_padded)

    y_scaled = pl.pallas_call(
        _ffn_body,
        grid_spec=_ffn_grid_spec,
        out_shape=jax.ShapeDtypeStruct((P_PAD, D), jnp.float32),
    )(blk, eid, x_padded,
      w0.astype(jnp.bfloat16), w1.astype(jnp.bfloat16),
      w2.astype(jnp.bfloat16), rw_padded[:, None])

    out_flat = _make_sc_row_gather(P, 128)(y_scaled, pos)
    return out_flat.reshape(T, K, D)


# speedup vs baseline: 1.4116x; 1.2397x over previous
import functools

import jax
import jax.numpy as jnp
from jax import lax
from jax.experimental import pallas as pl
from jax.experimental.pallas import tpu as pltpu
from jax.experimental.pallas import tpu_sc as plsc

T = 2048
D = 768
F = 2048
E = 8
K = 2

B = 256
P = T * K
P_PAD = P + E * B
G = P_PAD // B

NC = 2
NS = 16
NW = NC * NS


@functools.lru_cache(maxsize=None)
def _make_sc_row_gather(n_rows: int, chunk: int):
    n_per_w = n_rows // NW
    n_chunks = n_per_w // chunk
    assert n_per_w % chunk == 0 and chunk % 8 == 0

    mesh = plsc.VectorSubcoreMesh(core_axis_name="c", subcore_axis_name="s",
                                  num_cores=NC, num_subcores=NS)

    @functools.partial(
        pl.kernel,
        mesh=mesh,
        out_type=jax.ShapeDtypeStruct((n_rows, D), jnp.float32),
        scratch_types=[
            pltpu.VMEM((chunk,), jnp.int32),
            pltpu.VMEM((chunk, D), jnp.float32),
            pltpu.SemaphoreType.DMA,
        ],
    )
    def gather_kernel(table_hbm, idx_hbm, out_hbm, idx_v, rows_v, sem):
        wid = lax.axis_index("s") * NC + lax.axis_index("c")
        base = wid * n_per_w
        for c in range(n_chunks):
            off = base + c * chunk
            pltpu.sync_copy(idx_hbm.at[pl.ds(off, chunk)], idx_v)
            pltpu.async_copy(table_hbm.at[idx_v], rows_v, sem).wait()
            pltpu.sync_copy(rows_v, out_hbm.at[pl.ds(off, chunk)])

    return gather_kernel


def _ffn_body(blk_ref, eid_ref, x_ref, w0_ref, w1_ref, w2_ref, rw_ref, o_ref):
    s = pl.program_id(0)

    @pl.when(blk_ref[s] == s)
    def _():
        xb = x_ref[...]
        a = jnp.dot(xb, w0_ref[0], preferred_element_type=jnp.float32,
                    precision=lax.Precision.DEFAULT)
        b = jnp.dot(xb, w1_ref[0], preferred_element_type=jnp.float32,
                    precision=lax.Precision.DEFAULT)
        h = (a * jax.nn.sigmoid(a)) * b
        y = jnp.dot(h, w2_ref[0], preferred_element_type=jnp.float32,
                    precision=lax.Precision.DEFAULT)
        o_ref[...] = y * rw_ref[...]


_ffn_grid_spec = pltpu.PrefetchScalarGridSpec(
    num_scalar_prefetch=2,
    grid=(G,),
    in_specs=[
        pl.BlockSpec((B, D), lambda s, blk, eid: (blk[s], 0)),
        pl.BlockSpec((1, D, F), lambda s, blk, eid: (eid[s], 0, 0)),
        pl.BlockSpec((1, D, F), lambda s, blk, eid: (eid[s], 0, 0)),
        pl.BlockSpec((1, F, D), lambda s, blk, eid: (eid[s], 0, 0)),
        pl.BlockSpec((B, 1), lambda s, blk, eid: (blk[s], 0)),
    ],
    out_specs=pl.BlockSpec((B, D), lambda s, blk, eid: (blk[s], 0)),
)


def kernel(x, w0, w1, w2, selected_experts, routing_weights):
    e_flat = selected_experts.reshape(P).astype(jnp.int32)
    rw_flat = routing_weights.reshape(P)

    order = jnp.argsort(e_flat, stable=True).astype(jnp.int32)
    sorted_e = jnp.take(e_flat, order)
    counts = jnp.bincount(e_flat, length=E).astype(jnp.int32)
    csum = jnp.cumsum(counts)
    group_start = csum - counts
    padded_counts = ((counts + B - 1) // B) * B
    pcsum = jnp.cumsum(padded_counts)
    pad_start = pcsum - padded_counts
    pp = (jnp.arange(P, dtype=jnp.int32)
          - jnp.take(group_start, sorted_e) + jnp.take(pad_start, sorted_e))
    tok_padded = (jnp.arange(P_PAD, dtype=jnp.int32) % T).at[pp].set(order // K)
    pos = jnp.zeros((P,), jnp.int32).at[order].set(pp)
    rw_padded = jnp.zeros((P_PAD,), jnp.float32).at[pp].set(
        jnp.take(rw_flat, order))

    nb = pcsum[-1] // B
    s_ids = jnp.arange(G, dtype=jnp.int32)
    blk = jnp.minimum(s_ids, nb - 1).astype(jnp.int32)
    eid = jnp.searchsorted(pcsum, blk * B, side="right").astype(jnp.int32)

    x_padded = _make_sc_row_gather(P_PAD, 96)(x, tok_padded)

    y_scaled = pl.pallas_call(
        _ffn_body,
        grid_spec=_ffn_grid_spec,
        out_shape=jax.ShapeDtypeStruct((P_PAD, D), jnp.float32),
    )(blk, eid, x_padded, w0, w1, w2, rw_padded[:, None])

    out_flat = _make_sc_row_gather(P, 128)(y_scaled, pos)
    return out_flat.reshape(T, K, D)
